# baseline (speedup 1.0000x reference)
import functools

import jax
import jax.numpy as jnp
from jax import lax
from jax.experimental import pallas as pl
from jax.experimental.pallas import tpu as pltpu
from jax.experimental.pallas import tpu_sc as plsc

NC = 2
NS = 16
NW = NC * NS
CHUNK = 128


NB = 4


def _sc_gather(flat_idx, emb2f, emb1f):
    C = flat_idx.shape[1]
    R = NW * C
    K = emb2f.shape[1]

    mesh = plsc.VectorSubcoreMesh(core_axis_name="c", subcore_axis_name="s")

    @functools.partial(
        pl.kernel,
        mesh=mesh,
        compiler_params=pltpu.CompilerParams(use_tc_tiling_on_sc=False),
        out_type=[
            jax.ShapeDtypeStruct((R, CHUNK, K), jnp.float32),
            jax.ShapeDtypeStruct((NW, C, CHUNK), jnp.float32),
        ],
        scratch_types=[
            pltpu.VMEM((C, CHUNK), jnp.int32),
            pltpu.VMEM((NB, CHUNK, K), jnp.float32),
            pltpu.VMEM((C, CHUNK), jnp.float32),
            pltpu.SemaphoreType.DMA,
            pltpu.SemaphoreType.DMA,
        ],
    )
    def sc_kernel(idx_hbm, emb2_hbm, emb1_hbm, g2_hbm, g1_hbm,
                  idx_v, ring_v, r1_v, sem2, sem1):
        wid = lax.axis_index("s") * NC + lax.axis_index("c")
        base = wid * C
        pltpu.sync_copy(idx_hbm.at[wid], idx_v)

        def g2_start(j):
            pltpu.make_async_copy(
                emb2_hbm.at[idx_v.at[j]], ring_v.at[lax.rem(j, NB)], sem2).start()

        def g2_finish(j):
            pltpu.make_async_copy(
                emb2_hbm.at[idx_v.at[j]], ring_v.at[lax.rem(j, NB)], sem2).wait()
            pltpu.sync_copy(ring_v.at[lax.rem(j, NB)], g2_hbm.at[base + j])

        def fire1(j, carry):
            pltpu.make_async_copy(emb1_hbm.at[idx_v.at[j]], r1_v.at[j], sem1).start()
            return carry

        lax.fori_loop(0, C, fire1, 0)

        def prime(j, carry):
            g2_start(j)
            return carry

        lax.fori_loop(0, NB, prime, 0)

        def steady(j, carry):
            g2_finish(j - NB)
            g2_start(j)
            return carry

        lax.fori_loop(NB, C, steady, 0)

        def tail(j, carry):
            g2_finish(j)
            return carry

        lax.fori_loop(C - NB, C, tail, 0)

        def drain1(j, carry):
            pltpu.make_async_copy(emb1_hbm.at[idx_v.at[j]], r1_v.at[j], sem1).wait()
            return carry

        lax.fori_loop(0, C, drain1, 0)
        pltpu.sync_copy(r1_v, g1_hbm.at[wid])

    return sc_kernel(flat_idx, emb2f, emb1f)


def _tc_forward(g2r, g1r, Xv, W0, b0, W1, b1, W2, b2, bias, *, BB):
    B, FK = g2r.shape
    F = Xv.shape[1]
    K = FK // F
    H = W0.shape[1]
    grid = B // BB

    def body(g2_ref, g1_ref, xv_ref, w0_ref, b0_ref, w1_ref, b1_ref,
             w2_ref, b2_ref, bias_ref, out_ref, emb_ref):
        xv = xv_ref[...]
        summed = jnp.zeros((BB, K), jnp.float32)
        sumsq = jnp.zeros((BB, K), jnp.float32)
        for f in range(F):
            e = g2_ref[:, f * K:(f + 1) * K] * xv[:, f:f + 1]
            emb_ref[:, f * K:(f + 1) * K] = e
            summed = summed + e
            sumsq = sumsq + e * e
        fm2 = 0.5 * jnp.sum(summed * summed - sumsq, axis=1, keepdims=True)
        fm1 = jnp.sum(g1_ref[...] * xv, axis=1, keepdims=True)
        h = jnp.dot(emb_ref[...], w0_ref[...], preferred_element_type=jnp.float32)
        h = jnp.maximum(h + b0_ref[...], 0.0)
        h = jnp.dot(h, w1_ref[...], preferred_element_type=jnp.float32)
        h = jnp.maximum(h + b1_ref[...], 0.0)
        h = jnp.dot(h, w2_ref[...], preferred_element_type=jnp.float32)
        h = jnp.maximum(h + b2_ref[...], 0.0)
        deep = jnp.sum(h, axis=1, keepdims=True)
        out_ref[...] = fm1 + fm2 + deep + bias_ref[0, 0]

    return pl.pallas_call(
        body,
        grid=(grid,),
        in_specs=[
            pl.BlockSpec((BB, FK), lambda i: (i, 0)),
            pl.BlockSpec((BB, F), lambda i: (i, 0)),
            pl.BlockSpec((BB, F), lambda i: (i, 0)),
            pl.BlockSpec((FK, H), lambda i: (0, 0)),
            pl.BlockSpec((1, H), lambda i: (0, 0)),
            pl.BlockSpec((H, H), lambda i: (0, 0)),
            pl.BlockSpec((1, H), lambda i: (0, 0)),
            pl.BlockSpec((H, H), lambda i: (0, 0)),
            pl.BlockSpec((1, H), lambda i: (0, 0)),
            pl.BlockSpec(memory_space=pltpu.SMEM),
        ],
        out_specs=pl.BlockSpec((BB, 1), lambda i: (i, 0)),
        out_shape=jax.ShapeDtypeStruct((B, 1), jnp.float32),
        scratch_shapes=[pltpu.VMEM((BB, FK), jnp.float32)],
    )(g2r, g1r, Xv, W0, b0, W1, b1, W2, b2, bias)


def kernel(Xi, Xv, emb1, emb2, W0, b0, W1, b1, W2, b2, bias):
    B, F = Xv.shape
    V = emb1.shape[1]
    K = emb2.shape[2]
    H = W0.shape[1]

    flat_idx = (Xi[:, :, 0] + (jnp.arange(F, dtype=jnp.int32) * V)[None, :])
    flat_idx = flat_idx.reshape(NW, B * F // (CHUNK * NW), CHUNK)
    emb2f = emb2.reshape(F * V, K)
    emb1f = emb1.reshape(F * V)

    g2, g1 = _sc_gather(flat_idx, emb2f, emb1f)
    g2r = g2.reshape(B, F * K)
    g1r = g1.reshape(B, F)

    out = _tc_forward(g2r, g1r, Xv, W0, b0.reshape(1, H), W1, b1.reshape(1, H),
                      W2, b2.reshape(1, H), bias.reshape(1, 1), BB=256)
    return out.reshape(B)


# speedup vs baseline: 1.0150x; 1.0150x over previous
import functools

import jax
import jax.numpy as jnp
from jax import lax
from jax.experimental import pallas as pl
from jax.experimental.pallas import tpu as pltpu
from jax.experimental.pallas import tpu_sc as plsc

NC = 2
NS = 16
NW = NC * NS
CHUNK = 128


NB = 4


def _sc_gather(flat_idx, emb2f, emb1f):
    C = flat_idx.shape[1]
    R = NW * C
    K = emb2f.shape[1]

    mesh = plsc.VectorSubcoreMesh(core_axis_name="c", subcore_axis_name="s")

    @functools.partial(
        pl.kernel,
        mesh=mesh,
        compiler_params=pltpu.CompilerParams(use_tc_tiling_on_sc=False),
        out_type=[
            jax.ShapeDtypeStruct((R, CHUNK, K), jnp.float32),
            jax.ShapeDtypeStruct((NW, C, CHUNK), jnp.float32),
        ],
        scratch_types=[
            pltpu.VMEM((C, CHUNK), jnp.int32),
            pltpu.VMEM((NB, CHUNK, K), jnp.float32),
            pltpu.VMEM((C, CHUNK), jnp.float32),
            pltpu.SemaphoreType.DMA,
            pltpu.SemaphoreType.DMA,
        ],
    )
    def sc_kernel(idx_hbm, emb2_hbm, emb1_hbm, g2_hbm, g1_hbm,
                  idx_v, ring_v, r1_v, sem2, sem1):
        wid = lax.axis_index("s") * NC + lax.axis_index("c")
        base = wid * C
        pltpu.sync_copy(idx_hbm.at[wid], idx_v)

        def g2_start(j):
            pltpu.make_async_copy(
                emb2_hbm.at[idx_v.at[j]], ring_v.at[lax.rem(j, NB)], sem2).start()

        def g2_finish(j):
            pltpu.make_async_copy(
                emb2_hbm.at[idx_v.at[j]], ring_v.at[lax.rem(j, NB)], sem2).wait()
            pltpu.sync_copy(ring_v.at[lax.rem(j, NB)], g2_hbm.at[base + j])

        def fire1(j, carry):
            pltpu.make_async_copy(emb1_hbm.at[idx_v.at[j]], r1_v.at[j], sem1).start()
            return carry

        lax.fori_loop(0, C, fire1, 0)

        def prime(j, carry):
            g2_start(j)
            return carry

        lax.fori_loop(0, NB, prime, 0)

        def steady(j, carry):
            g2_finish(j - NB)
            g2_start(j)
            return carry

        lax.fori_loop(NB, C, steady, 0)

        def tail(j, carry):
            g2_finish(j)
            return carry

        lax.fori_loop(C - NB, C, tail, 0)

        def drain1(j, carry):
            pltpu.make_async_copy(emb1_hbm.at[idx_v.at[j]], r1_v.at[j], sem1).wait()
            return carry

        lax.fori_loop(0, C, drain1, 0)
        pltpu.sync_copy(r1_v, g1_hbm.at[wid])

    return sc_kernel(flat_idx, emb2f, emb1f)


def _tc_forward(g2r, g1r, Xv, Xve, W0s, b0, W1, b1, W2, b2, bias, *, BB, H, K):
    B, FK = g2r.shape
    F = Xv.shape[1]
    HS = W0s.shape[1]
    grid = B // BB

    def body(g2_ref, g1_ref, xv_ref, xve_ref, w0s_ref, b0_ref, w1_ref,
             b1_ref, w2_ref, b2_ref, bias_ref, out_ref):
        e = g2_ref[...] * xve_ref[...]
        hs = jnp.dot(e, w0s_ref[...], preferred_element_type=jnp.float32)
        summed = hs[:, H:]
        fm2 = 0.5 * (jnp.sum(summed * summed, axis=1, keepdims=True)
                     - jnp.sum(e * e, axis=1, keepdims=True))
        fm1 = jnp.sum(g1_ref[...] * xv_ref[...], axis=1, keepdims=True)
        h = jnp.maximum(hs[:, :H] + b0_ref[...], 0.0)
        h = jnp.dot(h, w1_ref[...], preferred_element_type=jnp.float32)
        h = jnp.maximum(h + b1_ref[...], 0.0)
        h = jnp.dot(h, w2_ref[...], preferred_element_type=jnp.float32)
        h = jnp.maximum(h + b2_ref[...], 0.0)
        deep = jnp.sum(h, axis=1, keepdims=True)
        out_ref[...] = fm1 + fm2 + deep + bias_ref[0, 0]

    return pl.pallas_call(
        body,
        grid=(grid,),
        in_specs=[
            pl.BlockSpec((BB, FK), lambda i: (i, 0)),
            pl.BlockSpec((BB, F), lambda i: (i, 0)),
            pl.BlockSpec((BB, F), lambda i: (i, 0)),
            pl.BlockSpec((BB, FK), lambda i: (i, 0)),
            pl.BlockSpec((FK, HS), lambda i: (0, 0)),
            pl.BlockSpec((1, H), lambda i: (0, 0)),
            pl.BlockSpec((H, H), lambda i: (0, 0)),
            pl.BlockSpec((1, H), lambda i: (0, 0)),
            pl.BlockSpec((H, H), lambda i: (0, 0)),
            pl.BlockSpec((1, H), lambda i: (0, 0)),
            pl.BlockSpec(memory_space=pltpu.SMEM),
        ],
        out_specs=pl.BlockSpec((BB, 1), lambda i: (i, 0)),
        out_shape=jax.ShapeDtypeStruct((B, 1), jnp.float32),
    )(g2r, g1r, Xv, Xve, W0s, b0, W1, b1, W2, b2, bias)


def kernel(Xi, Xv, emb1, emb2, W0, b0, W1, b1, W2, b2, bias):
    B, F = Xv.shape
    V = emb1.shape[1]
    K = emb2.shape[2]
    H = W0.shape[1]

    flat_idx = (Xi[:, :, 0] + (jnp.arange(F, dtype=jnp.int32) * V)[None, :])
    flat_idx = flat_idx.reshape(NW, B * F // (CHUNK * NW), CHUNK)
    emb2f = emb2.reshape(F * V, K)
    emb1f = emb1.reshape(F * V)

    g2, g1 = _sc_gather(flat_idx, emb2f, emb1f)
    g2r = g2.reshape(B, F * K)
    g1r = g1.reshape(B, F)

    Xve = jnp.repeat(Xv, K, axis=1)
    S = jnp.tile(jnp.eye(K, dtype=jnp.float32), (F, 1))
    W0s = jnp.concatenate([W0, S], axis=1)

    out = _tc_forward(g2r, g1r, Xv, Xve, W0s, b0.reshape(1, H),
                      W1, b1.reshape(1, H), W2, b2.reshape(1, H),
                      bias.reshape(1, 1), BB=256, H=H, K=K)
    return out.reshape(B)
